# Initial kernel scaffold; baseline (speedup 1.0000x reference)
import jax
import jax.numpy as jnp
from jax.experimental import pallas as pl


def kernel(input_emb, W, att_src, att_dst, bias):
    raise NotImplementedError("write your pallas kernel here")


# speedup vs baseline: 490.6412x; 490.6412x over previous
import numpy as np
import jax
import jax.numpy as jnp
from jax.experimental import pallas as pl

_N = 512
_D = 128
_H = 4
_F = 64


def _build_edge_index() -> np.ndarray:
    base = np.arange(_N - 1, dtype=np.int32)[None, :]
    src_col = np.arange(_N, dtype=np.int32)[:, None]
    dst = (base + (base >= src_col).astype(np.int32)).reshape(-1)
    src = np.repeat(np.arange(_N, dtype=np.int32), _N - 1)
    return np.stack([src, dst])


_EDGE_INDEX = _build_edge_index()


def _gat_adj_kernel(emb_ref, w_ref, asrc_ref, adst_ref, out_ref):
    emb = emb_ref[:]
    row = jax.lax.broadcasted_iota(jnp.int32, (_N, _N), 0)
    col = jax.lax.broadcasted_iota(jnp.int32, (_N, _N), 1)
    diag = row == col

    acc = jnp.zeros((_N, _N), dtype=jnp.float32)
    for h in range(_H):
        wh = w_ref[:, h * _F:(h + 1) * _F]
        xh = jax.lax.dot_general(
            emb, wh, (((1,), (0,)), ((), ())),
            preferred_element_type=jnp.float32,
            precision=jax.lax.Precision.HIGHEST,
        )
        s = jax.lax.dot_general(
            xh, asrc_ref[h:h + 1, :], (((1,), (1,)), ((), ())),
            preferred_element_type=jnp.float32,
            precision=jax.lax.Precision.HIGHEST,
        )
        d = jax.lax.dot_general(
            adst_ref[h:h + 1, :], xh, (((1,), (1,)), ((), ())),
            preferred_element_type=jnp.float32,
            precision=jax.lax.Precision.HIGHEST,
        )
        e = s + d
        e = jnp.where(e >= 0, e, 0.2 * e)
        masked = jnp.where(diag, -jnp.inf, e)
        amax = jnp.max(masked, axis=0, keepdims=True)
        p = jnp.where(diag, 0.0, jnp.exp(e - amax))
        denom = jnp.sum(p, axis=0, keepdims=True) + 1e-16
        acc = acc + p / denom
    out_ref[:] = acc * (1.0 / _H)


@jax.jit
def _adj(input_emb, W, att_src, att_dst):
    return pl.pallas_call(
        _gat_adj_kernel,
        out_shape=jax.ShapeDtypeStruct((_N, _N), jnp.float32),
    )(input_emb, W, att_src, att_dst)


def kernel(input_emb, W, att_src, att_dst, bias):
    del bias
    edge_index = jnp.asarray(_EDGE_INDEX)
    adj_matrix = _adj(input_emb, W, att_src, att_dst)
    return (edge_index, adj_matrix)
